# baseline (speedup 1.0000x reference)
import functools

import jax
import jax.numpy as jnp
from jax import lax
from jax.experimental import pallas as pl
from jax.experimental.pallas import tpu as pltpu
from jax.experimental.pallas import tpu_sc as plsc

_info = plsc.get_sparse_core_info()
_NC, _NS = _info.num_cores, _info.num_subcores
_NW = _NC * _NS

_DB = 8
_VP = 1024


@functools.partial(jax.jit, static_argnums=(2, 3, 4))
def _embed_t(Wt, xT, B, L, d):
    nd = d // _DB
    ng = _NW // nd
    base = L // ng
    rem = L % ng
    mesh = plsc.VectorSubcoreMesh(core_axis_name="c", subcore_axis_name="s")
    assert base >= 2

    @functools.partial(
        pl.kernel,
        mesh=mesh,
        out_type=jax.ShapeDtypeStruct((L, d, B), jnp.float32),
        scratch_types=[
            pltpu.VMEM((_DB * _VP,), jnp.float32),
            pltpu.VMEM((2, 1, B + 16), jnp.int32),
            pltpu.VMEM((2, 1, _DB, B), jnp.float32),
            pltpu.SemaphoreType.DMA,
            pltpu.SemaphoreType.DMA,
            pltpu.SemaphoreType.DMA,
        ],
        compiler_params=pltpu.CompilerParams(needs_layout_passes=False),
    )
    def k(wt_hbm, xt_hbm, out_hbm, wt_v, idx_v, out_v, sem_t, sem_i, sem_o):
        wid = lax.axis_index("s") * _NC + lax.axis_index("c")
        dt = lax.rem(wid, nd)
        g = wid // nd
        l0 = g * base + jnp.minimum(g, rem)
        l1 = l0 + base + jnp.where(g < rem, 1, 0)

        bases = [jnp.full((16,), d8 * _VP, jnp.int32) for d8 in range(_DB)]

        def start_idx(l, p):
            pltpu.async_copy(
                xt_hbm.at[pl.ds(l, 1)], idx_v.at[p, :, pl.ds(0, B)], sem_i
            )

        def wait_idx():
            pltpu.make_async_copy(
                xt_hbm.at[pl.ds(0, 1)], idx_v.at[0, :, pl.ds(0, B)], sem_i
            ).wait()

        def start_out(l, p):
            pltpu.async_copy(
                out_v.at[p],
                out_hbm.at[
                    pl.ds(l, 1),
                    pl.ds(pl.multiple_of(dt * _DB, _DB), _DB),
                ],
                sem_o,
            )

        def wait_out():
            pltpu.make_async_copy(
                out_v.at[0], out_hbm.at[pl.ds(0, 1), pl.ds(0, _DB)], sem_o
            ).wait()

        tbl = pltpu.async_copy(
            wt_hbm.at[pl.ds(pl.multiple_of(dt * _DB * _VP, 8), _DB * _VP)],
            wt_v,
            sem_t,
        )
        start_idx(l0, 0)
        tbl.wait()

        def unit(l, _):
            p = lax.rem(l - l0, 2)

            @pl.when(l + 1 < l1)
            def _():
                start_idx(l + 1, 1 - p)

            wait_idx()

            @pl.when(l - l0 >= 2)
            def _():
                wait_out()

            def addrs_of(iv):
                return tuple(iv + bases[d8] for d8 in range(_DB))

            def col(cg, carry):
                addrs, vals = carry
                nxt = idx_v[p, 0, pl.ds(cg * 16 + 16, 16)]
                new = tuple(
                    plsc.load_gather(wt_v, [addrs[d8]]) for d8 in range(_DB)
                )
                for d8 in range(_DB):
                    out_v[p, 0, d8, pl.ds(cg * 16 - 16, 16)] = vals[d8]
                return addrs_of(nxt), new

            iv0 = idx_v[p, 0, pl.ds(0, 16)]
            vals0 = tuple(
                plsc.load_gather(wt_v, [a]) for a in addrs_of(iv0)
            )
            iv1 = idx_v[p, 0, pl.ds(16, 16)]
            _, last = lax.fori_loop(
                1, B // 16, col, (addrs_of(iv1), vals0), unroll=3
            )
            for d8 in range(_DB):
                out_v[p, 0, d8, pl.ds(B - 16, 16)] = last[d8]
            start_out(l, p)
            return ()

        lax.fori_loop(l0, l1, unit, ())
        wait_out()
        wait_out()

    return k(Wt, xT)


def kernel(x, W):
    B, L = x.shape
    V, D = W.shape
    Wt = jnp.pad(W.T, ((0, 0), (0, _VP - V))).reshape(-1)
    out_t = _embed_t(Wt, x.T, B, L, D)
    return jnp.transpose(out_t, (2, 0, 1))


# speedup vs baseline: 1.0253x; 1.0253x over previous
import functools

import jax
import jax.numpy as jnp
from jax import lax
from jax.experimental import pallas as pl
from jax.experimental.pallas import tpu as pltpu
from jax.experimental.pallas import tpu_sc as plsc

_info = plsc.get_sparse_core_info()
_NC, _NS = _info.num_cores, _info.num_subcores
_NW = _NC * _NS

_DB = 8
_VP = 1024


@functools.partial(jax.jit, static_argnums=(2, 3, 4))
def _embed_t(Wt, xT, B, L, d):
    nd = d // _DB
    ng = _NW // nd
    base = L // ng
    rem = L % ng
    mesh = plsc.VectorSubcoreMesh(core_axis_name="c", subcore_axis_name="s")
    assert base >= 2

    @functools.partial(
        pl.kernel,
        mesh=mesh,
        out_type=jax.ShapeDtypeStruct((L, d, B), jnp.float32),
        scratch_types=[
            pltpu.VMEM((_DB * _VP,), jnp.float32),
            pltpu.VMEM((2, 1, B + 16), jnp.int32),
            pltpu.VMEM((2, 1, _DB, B), jnp.float32),
            pltpu.SemaphoreType.DMA,
            pltpu.SemaphoreType.DMA,
            pltpu.SemaphoreType.DMA,
        ],
        compiler_params=pltpu.CompilerParams(needs_layout_passes=False),
    )
    def k(wt_hbm, xt_hbm, out_hbm, wt_v, idx_v, out_v, sem_t, sem_i, sem_o):
        wid = lax.axis_index("s") * _NC + lax.axis_index("c")
        dt = lax.rem(wid, nd)
        g = wid // nd
        l0 = g * base + jnp.minimum(g, rem)
        l1 = l0 + base + jnp.where(g < rem, 1, 0)

        bases = [jnp.full((16,), d8 * _VP, jnp.int32) for d8 in range(_DB)]

        def start_idx(l, p):
            pltpu.async_copy(
                xt_hbm.at[pl.ds(l, 1)], idx_v.at[p, :, pl.ds(0, B)], sem_i
            )

        def wait_idx():
            pltpu.make_async_copy(
                xt_hbm.at[pl.ds(0, 1)], idx_v.at[0, :, pl.ds(0, B)], sem_i
            ).wait()

        def start_out(l, p):
            pltpu.async_copy(
                out_v.at[p],
                out_hbm.at[
                    pl.ds(l, 1),
                    pl.ds(pl.multiple_of(dt * _DB, _DB), _DB),
                ],
                sem_o,
            )

        def wait_out():
            pltpu.make_async_copy(
                out_v.at[0], out_hbm.at[pl.ds(0, 1), pl.ds(0, _DB)], sem_o
            ).wait()

        tbl = pltpu.async_copy(
            wt_hbm.at[pl.ds(pl.multiple_of(dt * _DB * _VP, 8), _DB * _VP)],
            wt_v,
            sem_t,
        )
        start_idx(l0, 0)
        tbl.wait()

        def unit(l, _):
            p = lax.rem(l - l0, 2)

            @pl.when(l + 1 < l1)
            def _():
                start_idx(l + 1, 1 - p)

            wait_idx()

            @pl.when(l - l0 >= 2)
            def _():
                wait_out()

            def addrs_of(iv):
                return tuple(iv + bases[d8] for d8 in range(_DB))

            def col(cg, carry):
                addrs, vals = carry
                nxt = idx_v[p, 0, pl.ds(cg * 16 + 16, 16)]
                new = tuple(
                    plsc.load_gather(wt_v, [addrs[d8]]) for d8 in range(_DB)
                )
                for d8 in range(_DB):
                    out_v[p, 0, d8, pl.ds(cg * 16 - 16, 16)] = vals[d8]
                return addrs_of(nxt), new

            iv0 = idx_v[p, 0, pl.ds(0, 16)]
            vals0 = tuple(
                plsc.load_gather(wt_v, [a]) for a in addrs_of(iv0)
            )
            iv1 = idx_v[p, 0, pl.ds(16, 16)]
            _, last = lax.fori_loop(
                1, B // 16, col, (addrs_of(iv1), vals0), unroll=8
            )
            for d8 in range(_DB):
                out_v[p, 0, d8, pl.ds(B - 16, 16)] = last[d8]
            start_out(l, p)
            return ()

        lax.fori_loop(l0, l1, unit, ())
        wait_out()
        wait_out()

    return k(Wt, xT)


def kernel(x, W):
    B, L = x.shape
    V, D = W.shape
    Wt = jnp.pad(W.T, ((0, 0), (0, _VP - V))).reshape(-1)
    out_t = _embed_t(Wt, x.T, B, L, D)
    return jnp.transpose(out_t, (2, 0, 1))
